# baseline (speedup 1.0000x reference)
import functools

import jax
import jax.numpy as jnp
from jax import lax
from jax.experimental import pallas as pl
from jax.experimental.pallas import tpu as pltpu
from jax.experimental.pallas import tpu_sc as plsc

DIM = 128
K = 100000
BATCH = 16384

NC = 2
RPW = 8
CW = 3840
NFULL = 21
LASTW = K - BATCH - NFULL * CW
NL = 10

TBLK = 2048
NTBLK = BATCH // TBLK


def _sc_copy_body(kt_hbm, q_hbm, o_hbm, buf, lastbuf, rembuf, isem, osem, lsem):
    ch = lax.axis_index("c")
    r0 = pl.multiple_of(lax.axis_index("s") * RPW, RPW)

    def _co(l):
        return pl.multiple_of(BATCH + (2 * l + ch) * CW, 128)

    def din(l):
        return pltpu.make_async_copy(
            q_hbm.at[pl.ds(r0, RPW), pl.ds(_co(l), CW)], buf.at[l % 2],
            isem.at[l % 2])

    def dout(l):
        return pltpu.make_async_copy(
            buf.at[l % 2], o_hbm.at[pl.ds(r0, RPW), pl.ds(_co(l), CW)],
            osem.at[l % 2])

    def lin():
        return pltpu.make_async_copy(
            q_hbm.at[pl.ds(r0, RPW), pl.ds(K - LASTW, LASTW)],
            lastbuf, lsem.at[0])

    def lout():
        return pltpu.make_async_copy(
            lastbuf, o_hbm.at[pl.ds(r0, RPW), pl.ds(K - LASTW, LASTW)],
            lsem.at[1])

    din(0).start()
    for l in range(NL):
        if l + 1 < NL:
            if l >= 1:
                dout(l - 1).wait()
            din(l + 1).start()
        din(l).wait()
        dout(l).start()
    dout(NL - 2).wait()

    @pl.when(ch == 0)
    def _():
        din(NL).start()
        din(NL).wait()
        dout(NL).start()
        dout(NL).wait()

    @pl.when(ch == 1)
    def _():
        lin().start()
        lin().wait()
        lout().start()
        lout().wait()

    dout(NL - 1).wait()

    def hco(m):
        return pl.multiple_of((2 * m + ch) * CW, 128)

    def hin(m):
        return pltpu.make_async_copy(
            kt_hbm.at[pl.ds(r0, RPW), pl.ds(hco(m), CW)], buf.at[m % 2],
            isem.at[m % 2])

    def hout(m):
        return pltpu.make_async_copy(
            buf.at[m % 2], o_hbm.at[pl.ds(r0, RPW), pl.ds(hco(m), CW)],
            osem.at[m % 2])

    def rin():
        return pltpu.make_async_copy(
            kt_hbm.at[pl.ds(r0, RPW), pl.ds(4 * CW, BATCH - 4 * CW)],
            rembuf, lsem.at[0])

    def rout():
        return pltpu.make_async_copy(
            rembuf, o_hbm.at[pl.ds(r0, RPW), pl.ds(4 * CW, BATCH - 4 * CW)],
            lsem.at[1])

    hin(0).start()
    hin(1).start()

    @pl.when(ch == 0)
    def _():
        rin().start()

    hin(0).wait()
    hout(0).start()
    hin(1).wait()
    hout(1).start()

    @pl.when(ch == 0)
    def _():
        rin().wait()
        rout().start()
        rout().wait()

    hout(0).wait()
    hout(1).wait()


_sc_copy = functools.partial(
    pl.kernel,
    out_type=jax.ShapeDtypeStruct((DIM, K), jnp.float32),
    mesh=plsc.VectorSubcoreMesh(core_axis_name="c", subcore_axis_name="s"),
    scratch_types=[
        pltpu.VMEM((2, RPW, CW), jnp.float32),
        pltpu.VMEM((RPW, LASTW), jnp.float32),
        pltpu.VMEM((RPW, BATCH - 4 * CW), jnp.float32),
        pltpu.SemaphoreType.DMA((2,)),
        pltpu.SemaphoreType.DMA((2,)),
        pltpu.SemaphoreType.DMA((2,)),
    ],
)(_sc_copy_body)


def _xpose_body(k_ref, o_ref):
    o_ref[...] = k_ref[...].T


def kernel(keys, queue):
    keys_t = pl.pallas_call(
        _xpose_body,
        grid=(NTBLK,),
        in_specs=[pl.BlockSpec((TBLK, DIM), lambda i: (i, 0))],
        out_specs=pl.BlockSpec((DIM, TBLK), lambda i: (0, i)),
        out_shape=jax.ShapeDtypeStruct((DIM, BATCH), jnp.float32),
    )(keys)

    new_queue = _sc_copy(keys_t, queue)
    new_ptr = jnp.array([BATCH % K], dtype=jnp.int32)
    return new_queue, new_ptr


# speedup vs baseline: 1.0523x; 1.0523x over previous
import functools

import jax
import jax.numpy as jnp
from jax import lax
from jax.experimental import pallas as pl
from jax.experimental.pallas import tpu as pltpu
from jax.experimental.pallas import tpu_sc as plsc

DIM = 128
K = 100000
BATCH = 16384

NC = 2
RPW = 8
CW = 3840
NFULL = 21
LASTW = K - BATCH - NFULL * CW
NL = 10

TBLK = 2048
NTBLK = BATCH // TBLK


def _sc_copy_body(q_hbm, o_hbm, buf, lastbuf, isem, osem, lsem):
    ch = lax.axis_index("c")
    r0 = pl.multiple_of(lax.axis_index("s") * RPW, RPW)

    def _co(l):
        return pl.multiple_of(BATCH + (2 * l + ch) * CW, 128)

    def din(l):
        return pltpu.make_async_copy(
            q_hbm.at[pl.ds(r0, RPW), pl.ds(_co(l), CW)], buf.at[l % 2],
            isem.at[l % 2])

    def dout(l):
        return pltpu.make_async_copy(
            buf.at[l % 2], o_hbm.at[pl.ds(r0, RPW), pl.ds(_co(l), CW)],
            osem.at[l % 2])

    def lin():
        return pltpu.make_async_copy(
            q_hbm.at[pl.ds(r0, RPW), pl.ds(K - LASTW, LASTW)],
            lastbuf, lsem.at[0])

    def lout():
        return pltpu.make_async_copy(
            lastbuf, o_hbm.at[pl.ds(r0, RPW), pl.ds(K - LASTW, LASTW)],
            lsem.at[1])

    din(0).start()
    for l in range(NL):
        if l + 1 < NL:
            if l >= 1:
                dout(l - 1).wait()
            din(l + 1).start()
        din(l).wait()
        dout(l).start()
    dout(NL - 2).wait()

    @pl.when(ch == 0)
    def _():
        din(NL).start()
        din(NL).wait()
        dout(NL).start()
        dout(NL).wait()

    @pl.when(ch == 1)
    def _():
        lin().start()
        lin().wait()
        lout().start()
        lout().wait()

    dout(NL - 1).wait()


_sc_copy = functools.partial(
    pl.kernel,
    out_type=jax.ShapeDtypeStruct((DIM, K), jnp.float32),
    mesh=plsc.VectorSubcoreMesh(core_axis_name="c", subcore_axis_name="s"),
    scratch_types=[
        pltpu.VMEM((2, RPW, CW), jnp.float32),
        pltpu.VMEM((RPW, LASTW), jnp.float32),
        pltpu.SemaphoreType.DMA((2,)),
        pltpu.SemaphoreType.DMA((2,)),
        pltpu.SemaphoreType.DMA((2,)),
    ],
)(_sc_copy_body)


def _xpose_body(k_ref, _, o_ref):
    o_ref[...] = k_ref[...].T


def kernel(keys, queue):
    tail = _sc_copy(queue)

    new_queue = pl.pallas_call(
        _xpose_body,
        grid=(NTBLK,),
        in_specs=[
            pl.BlockSpec((TBLK, DIM), lambda i: (i, 0)),
            pl.BlockSpec(memory_space=pl.ANY),
        ],
        out_specs=pl.BlockSpec((DIM, TBLK), lambda i: (0, i)),
        out_shape=jax.ShapeDtypeStruct((DIM, K), jnp.float32),
        input_output_aliases={1: 0},
    )(keys, tail)

    new_ptr = jnp.array([BATCH % K], dtype=jnp.int32)
    return new_queue, new_ptr
